# baseline (speedup 1.0000x reference)
import functools
import jax
import jax.numpy as jnp
from jax import lax
from jax.experimental import pallas as pl
from jax.experimental.pallas import tpu as pltpu
from jax.experimental.pallas import tpu_sc as plsc

_M = 256
_B = 128
_L = 16
_NW = 32
_SPW = _M // _NW
_SNBUF = 2
_YNBUF = 4


def _build_route(idx_v, route_smem, wid):
    for d in range(_SPW):
        route_smem[d] = jnp.int32(-1)

    def body(kq, c):
        chunk = idx_v[pl.ds(kq * _L, _L)]
        for i in range(_L):
            d = chunk[i] - wid * _SPW

            @pl.when((d >= 0) & (d < _SPW))
            def _(d=d, q=kq * _L + i):
                route_smem[d] = q
        return c
    lax.fori_loop(0, _B // _L, body, jnp.int32(0))


def _make_ring(chunks, bufs, sin, sout, nbuf, deff):
    total = len(chunks)

    def drain_in_start_out(cp):
        bp = cp % nbuf
        _, _, mem_src, dst = chunks[cp]
        pltpu.make_async_copy(mem_src, bufs.at[bp], sin[bp]).wait()
        pltpu.make_async_copy(bufs.at[bp], dst, sout[bp]).start()

    def step(c):
        j, val_src, mem_src, dst = chunks[c]
        b = c % nbuf
        if c >= nbuf:
            pltpu.make_async_copy(bufs.at[b], chunks[c - nbuf][3],
                                  sout[b]).wait()

        @pl.when(j >= 0)
        def _():
            pltpu.make_async_copy(val_src, bufs.at[b], sin[b]).start()

        @pl.when(j < 0)
        def _():
            pltpu.make_async_copy(mem_src, bufs.at[b], sin[b]).start()

        if c >= deff:
            drain_in_start_out(c - deff)

    def tail():
        for cp in range(max(total - deff, 0), total):
            drain_in_start_out(cp)
        for cp in range(max(total - nbuf, 0), total):
            bp = cp % nbuf
            pltpu.make_async_copy(bufs.at[bp], chunks[cp][3], sout[bp]).wait()

    return total, step, tail


def _sc_kernel(idx_hbm, m0, m1, m2, m3, m4, v0, v1, v2, v3, v4,
               o0, o1, o2, o3, o4, idx_v, s_in, route_smem):
    wid = lax.axis_index("s") * 2 + lax.axis_index("c")
    pltpu.sync_copy(idx_hbm, idx_v)
    _build_route(idx_v, route_smem, wid)

    small_mems = (m0, m1, m2, m4)
    small_vals = (v0, v1, v2, v4)
    small_outs = (o0, o1, o2, o4)

    slot_j = [route_smem[k] for k in range(_SPW)]
    slot_jj = [jnp.maximum(j, 0) for j in slot_j]
    slot_m = [wid * _SPW + k for k in range(_SPW)]

    chunks = []
    for k in range(_SPW):
        for t in range(4):
            chunks.append((slot_j[k],
                           small_vals[t].at[slot_jj[k]],
                           small_mems[t].at[slot_m[k]],
                           small_outs[t].at[slot_m[k]]))
        chunks.append((slot_j[k], v3.at[slot_jj[k]], m3.at[slot_m[k]],
                       o3.at[slot_m[k]]))

    nsem = len(s_in)
    for c, (j, val_src, mem_src, dst) in enumerate(chunks):
        sem = s_in[c % nsem]

        @pl.when(j >= 0)
        def _():
            pltpu.make_async_copy(val_src, dst, sem).start()

        @pl.when(j < 0)
        def _():
            pltpu.make_async_copy(mem_src, dst, sem).start()

    for c, (j, val_src, mem_src, dst) in enumerate(chunks):
        pltpu.make_async_copy(mem_src, dst, s_in[c % nsem]).wait()


def kernel(x_i_mem, y_j_mem, x_i_new_mem, y_j_new_mem, P_mem,
           x_i_val, y_j_val, x_i_new_val, y_j_new_val, P_val, idx):
    mems = (x_i_mem, y_j_mem, x_i_new_mem, y_j_new_mem,
            P_mem.reshape(_M, 20, 1000))
    vals = (x_i_val, y_j_val, x_i_new_val, y_j_new_val,
            P_val.reshape(_B, 20, 1000))

    mesh = plsc.VectorSubcoreMesh(core_axis_name="c", subcore_axis_name="s")
    k = functools.partial(
        pl.kernel,
        out_type=[jax.ShapeDtypeStruct(t.shape, t.dtype) for t in mems],
        mesh=mesh,
        scratch_types=[
            pltpu.VMEM((_B,), jnp.int32),
            [pltpu.SemaphoreType.DMA] * 4,
            pltpu.SMEM((_SPW,), jnp.int32),
        ],
    )(_sc_kernel)
    outs = k(idx, *mems, *vals)
    return (outs[0], outs[1], outs[2], outs[3], outs[4].reshape(P_mem.shape))


# speedup vs baseline: 16.8011x; 16.8011x over previous
import functools
import jax
import jax.numpy as jnp
from jax import lax
from jax.experimental import pallas as pl
from jax.experimental.pallas import tpu as pltpu
from jax.experimental.pallas import tpu_sc as plsc

_M = 256
_B = 128
_L = 16
_NW = 32
_SPW = _M // _NW
_SNBUF = 2
_YNBUF = 2
_YROWS = 2


def _build_route(idx_v, route_smem, wid):
    for d in range(_SPW):
        route_smem[d] = jnp.int32(-1)

    def body(kq, c):
        chunk = idx_v[pl.ds(kq * _L, _L)]
        for i in range(_L):
            d = chunk[i] - wid * _SPW

            @pl.when((d >= 0) & (d < _SPW))
            def _(d=d, q=kq * _L + i):
                route_smem[d] = q
        return c
    lax.fori_loop(0, _B // _L, body, jnp.int32(0))


def _make_ring(chunks, bufs, sin, sout, nbuf, deff):
    total = len(chunks)

    def drain_in_start_out(cp):
        bp = cp % nbuf
        _, _, mem_src, dst = chunks[cp]
        pltpu.make_async_copy(mem_src, bufs.at[bp], sin[bp]).wait()
        pltpu.make_async_copy(bufs.at[bp], dst, sout[bp]).start()

    def step(c):
        j, val_src, mem_src, dst = chunks[c]
        b = c % nbuf
        if c >= nbuf:
            pltpu.make_async_copy(bufs.at[b], chunks[c - nbuf][3],
                                  sout[b]).wait()

        @pl.when(j >= 0)
        def _():
            pltpu.make_async_copy(val_src, bufs.at[b], sin[b]).start()

        @pl.when(j < 0)
        def _():
            pltpu.make_async_copy(mem_src, bufs.at[b], sin[b]).start()

        if c >= deff:
            drain_in_start_out(c - deff)

    def tail():
        for cp in range(max(total - deff, 0), total):
            drain_in_start_out(cp)
        for cp in range(max(total - nbuf, 0), total):
            bp = cp % nbuf
            pltpu.make_async_copy(bufs.at[bp], chunks[cp][3], sout[bp]).wait()

    return total, step, tail


def _sc_kernel(idx_hbm, m0, m1, m2, m3, m4, v0, v1, v2, v3, v4,
               o0, o1, o2, o3, o4, idx_v, sbufs, ybufs,
               s_in, s_out, y_in, y_out, route_smem):
    wid = lax.axis_index("s") * 2 + lax.axis_index("c")
    pltpu.sync_copy(idx_hbm, idx_v)
    _build_route(idx_v, route_smem, wid)

    small_mems = (m0, m1, m2, m4)
    small_vals = (v0, v1, v2, v4)
    small_outs = (o0, o1, o2, o4)

    slot_j = [route_smem[k] for k in range(_SPW)]
    slot_jj = [jnp.maximum(j, 0) for j in slot_j]
    slot_m = [wid * _SPW + k for k in range(_SPW)]

    schunks = []
    for k in range(_SPW):
        for t in range(4):
            schunks.append((slot_j[k],
                            small_vals[t].at[slot_jj[k]],
                            small_mems[t].at[slot_m[k]],
                            small_outs[t].at[slot_m[k]]))
    ts, step_s, tail_s = _make_ring(schunks, sbufs, s_in, s_out, _SNBUF, 1)

    ychunks = []
    for k in range(_SPW):
        for c2 in range(20 // _YROWS):
            ychunks.append((slot_j[k],
                            v3.at[pl.ds(slot_jj[k] * 20 + c2 * _YROWS, _YROWS)],
                            m3.at[pl.ds(slot_m[k] * 20 + c2 * _YROWS, _YROWS)],
                            o3.at[pl.ds(slot_m[k] * 20 + c2 * _YROWS, _YROWS)]))
    ty, step_y, tail_y = _make_ring(ychunks, ybufs, y_in, y_out, _YNBUF, 1)

    ratio = ty // ts
    for c in range(ty):
        step_y(c)
        if c % ratio == ratio - 1 and c // ratio < ts:
            step_s(c // ratio)
    for c in range((ty // ratio), ts):
        step_s(c)
    tail_y()
    tail_s()


def kernel(x_i_mem, y_j_mem, x_i_new_mem, y_j_new_mem, P_mem,
           x_i_val, y_j_val, x_i_new_val, y_j_new_val, P_val, idx):
    mems = (x_i_mem, y_j_mem, x_i_new_mem,
            y_j_new_mem.reshape(_M * 20, 10, 1000), P_mem.reshape(_M, 20, 1000))
    vals = (x_i_val, y_j_val, x_i_new_val,
            y_j_new_val.reshape(_B * 20, 10, 1000), P_val.reshape(_B, 20, 1000))

    mesh = plsc.VectorSubcoreMesh(core_axis_name="c", subcore_axis_name="s")
    k = functools.partial(
        pl.kernel,
        out_type=[jax.ShapeDtypeStruct(t.shape, t.dtype) for t in mems],
        mesh=mesh,
        scratch_types=[
            pltpu.VMEM((_B,), jnp.int32),
            pltpu.VMEM((_SNBUF, 20, 1000), jnp.float32),
            pltpu.VMEM((_YNBUF, _YROWS, 10, 1000), jnp.float32),
            [pltpu.SemaphoreType.DMA] * _SNBUF,
            [pltpu.SemaphoreType.DMA] * _SNBUF,
            [pltpu.SemaphoreType.DMA] * _YNBUF,
            [pltpu.SemaphoreType.DMA] * _YNBUF,
            pltpu.SMEM((_SPW,), jnp.int32),
        ],
    )(_sc_kernel)
    outs = k(idx, *mems, *vals)
    return (outs[0], outs[1], outs[2],
            outs[3].reshape(y_j_new_mem.shape), outs[4].reshape(P_mem.shape))
